# Initial kernel scaffold; baseline (speedup 1.0000x reference)
import jax
import jax.numpy as jnp
from jax.experimental import pallas as pl


def kernel(query_seq, synth_set, topk):
    raise NotImplementedError("write your pallas kernel here")


# speedup vs baseline: 2.5429x; 2.5429x over previous
import functools

import jax
import jax.numpy as jnp
from jax import lax
from jax.experimental import pallas as pl
from jax.experimental.pallas import tpu as pltpu
from jax.experimental.pallas import tpu_sc as plsc

Q, S, D, K = 2048, 16384, 1024, 4
QB, SB = 256, 2048

NC, NS = 2, 16
NW = NC * NS
RPW = Q // NW
CR = 16
NCH = RPW // CR


def _topk_body(q_ref, s_ref, idx_ref, vals, idxs):
    si = pl.program_id(1)
    nsb = pl.num_programs(1)
    q = q_ref[...]
    s = s_ref[...]
    ab = lax.dot_general(q, s, (((1,), (1,)), ((), ())),
                         preferred_element_type=jnp.float32)
    na2 = jnp.sum(q * q, axis=1, keepdims=True)
    nb2 = jnp.sum(s * s, axis=1)[None, :]
    sq = jnp.maximum(na2 + nb2 - 2.0 * ab, 0.0)
    dp = (na2 + nb2 - sq) * 0.5
    score = dp / jnp.sqrt(na2 * nb2)

    @pl.when(si == 0)
    def _():
        vals[...] = jnp.full((QB, K), -jnp.inf, jnp.float32)
        idxs[...] = jnp.zeros((QB, K), jnp.int32)

    pos = lax.broadcasted_iota(jnp.int32, (QB, SB), 1)
    blk_v, blk_i = [], []
    work = score
    for _ in range(K):
        m = jnp.max(work, axis=1)
        am = jnp.min(jnp.where(work == m[:, None], pos, SB), axis=1)
        blk_v.append(m)
        blk_i.append(si * SB + am)
        work = jnp.where(pos == am[:, None], -jnp.inf, work)

    comb_v = jnp.concatenate(
        [vals[...]] + [v[:, None] for v in blk_v], axis=1)
    comb_i = jnp.concatenate(
        [idxs[...]] + [i[:, None] for i in blk_i], axis=1)
    pos8 = lax.broadcasted_iota(jnp.int32, (QB, 2 * K), 1)
    new_v, new_i = [], []
    for _ in range(K):
        m = jnp.max(comb_v, axis=1)
        am = jnp.min(jnp.where(comb_v == m[:, None], pos8, 2 * K), axis=1)
        gi = jnp.sum(jnp.where(pos8 == am[:, None], comb_i, 0), axis=1)
        new_v.append(m)
        new_i.append(gi)
        comb_v = jnp.where(pos8 == am[:, None], -jnp.inf, comb_v)
    vals[...] = jnp.concatenate([v[:, None] for v in new_v], axis=1)
    idxs[...] = jnp.concatenate([i[:, None] for i in new_i], axis=1)

    @pl.when(si == nsb - 1)
    def _():
        idx_ref[...] = idxs[...]


def _topk_indices(query_seq, synth_set):
    return pl.pallas_call(
        _topk_body,
        grid=(Q // QB, S // SB),
        in_specs=[
            pl.BlockSpec((QB, D), lambda qi, si: (qi, 0)),
            pl.BlockSpec((SB, D), lambda qi, si: (si, 0)),
        ],
        out_specs=pl.BlockSpec((QB, K), lambda qi, si: (qi, 0)),
        out_shape=jax.ShapeDtypeStruct((Q, K), jnp.int32),
        scratch_shapes=[
            pltpu.VMEM((QB, K), jnp.float32),
            pltpu.VMEM((QB, K), jnp.int32),
        ],
        compiler_params=pltpu.CompilerParams(
            dimension_semantics=("arbitrary", "arbitrary"),
        ),
    )(query_seq, synth_set)


def _gather_mean_body(synth_hbm, idx_hbm, out_hbm, idx_v, rows_v, out_v, sem):
    wid = lax.axis_index("s") * NC + lax.axis_index("c")
    for c in range(NCH):
        base_out = wid * RPW + c * CR
        pltpu.sync_copy(idx_hbm.at[pl.ds(base_out * K, CR * K)], idx_v)
        pltpu.async_copy(synth_hbm.at[idx_v], rows_v, sem).wait()

        def row_loop(i, _):
            def col_loop(dd, __):
                sl = pl.ds(dd * 16, 16)
                acc = (rows_v[K * i, sl] + rows_v[K * i + 1, sl]
                       + rows_v[K * i + 2, sl] + rows_v[K * i + 3, sl])
                out_v[i, sl] = acc * 0.25
                return __
            return lax.fori_loop(0, D // 16, col_loop, _)
        lax.fori_loop(0, CR, row_loop, 0)
        pltpu.sync_copy(out_v, out_hbm.at[pl.ds(base_out, CR)])


def _gather_mean(synth_set, flat_idx):
    mesh = plsc.VectorSubcoreMesh(core_axis_name="c", subcore_axis_name="s")
    f = functools.partial(
        pl.kernel,
        out_type=jax.ShapeDtypeStruct((Q, D), jnp.float32),
        mesh=mesh,
        scratch_types=[
            pltpu.VMEM((CR * K,), jnp.int32),
            pltpu.VMEM((CR * K, D), jnp.float32),
            pltpu.VMEM((CR, D), jnp.float32),
            pltpu.SemaphoreType.DMA,
        ],
    )(_gather_mean_body)
    return f(synth_set, flat_idx)


def kernel(query_seq, synth_set, topk):
    del topk
    idx = _topk_indices(query_seq, synth_set)
    return _gather_mean(synth_set, idx.reshape(Q * K))


# speedup vs baseline: 2.7398x; 1.0774x over previous
import functools

import jax
import jax.numpy as jnp
from jax import lax
from jax.experimental import pallas as pl
from jax.experimental.pallas import tpu as pltpu
from jax.experimental.pallas import tpu_sc as plsc

Q, S, D, K = 2048, 16384, 1024, 4
QB, SB = 256, 2048

NC, NS = 2, 16
NW = NC * NS
RPW = Q // NW
CR = 8
NCH = RPW // CR

def _topk_body(q_ref, s_ref, na_ref, nb_ref, idx_ref, dists, idxs):
    si = pl.program_id(0)
    nsb = pl.num_programs(0)
    qi = pl.program_id(1)
    q = q_ref[...]
    s = s_ref[...]
    ab = lax.dot_general(q, s, (((1,), (1,)), ((), ())),
                         preferred_element_type=jnp.float32)
    na = na_ref[...]
    nb = nb_ref[...]
    na2 = na ** 2
    nb2 = nb ** 2
    sq = jnp.maximum(na2 + nb2 - 2.0 * ab, 0.0)
    dotprod = (-sq + na2 + nb2) / 2.0
    d = 1.0 - dotprod / (na * nb)

    qslice = pl.ds(qi * QB, QB)

    @pl.when(si == 0)
    def _():
        dists[qslice, :] = jnp.full((QB, K), jnp.inf, jnp.float32)
        idxs[qslice, :] = jnp.zeros((QB, K), jnp.int32)

    pos = lax.broadcasted_iota(jnp.int32, (QB, SB), 1) + si * SB
    comb_d = jnp.concatenate([dists[qslice, :], d], axis=1)
    comb_i = jnp.concatenate([idxs[qslice, :], pos], axis=1)
    new_d, new_i = [], []
    for _ in range(K):
        m = jnp.min(comb_d, axis=1)
        gi = jnp.min(jnp.where(comb_d == m[:, None], comb_i, 2**31 - 1), axis=1)
        new_d.append(m)
        new_i.append(gi)
        comb_d = jnp.where(comb_i == gi[:, None], jnp.inf, comb_d)
    dists[qslice, :] = jnp.concatenate([v[:, None] for v in new_d], axis=1)
    final_i = jnp.concatenate([i[:, None] for i in new_i], axis=1)
    idxs[qslice, :] = final_i
    idx_ref[...] = final_i


def _topk_indices(query_seq, synth_set, na, nb):
    return pl.pallas_call(
        _topk_body,
        grid=(S // SB, Q // QB),
        in_specs=[
            pl.BlockSpec((QB, D), lambda si, qi: (qi, 0)),
            pl.BlockSpec((SB, D), lambda si, qi: (si, 0)),
            pl.BlockSpec((QB, 1), lambda si, qi: (qi, 0)),
            pl.BlockSpec((1, SB), lambda si, qi: (0, si)),
        ],
        out_specs=pl.BlockSpec((QB, K), lambda si, qi: (qi, 0)),
        out_shape=jax.ShapeDtypeStruct((Q, K), jnp.int32),
        scratch_shapes=[
            pltpu.VMEM((Q, K), jnp.float32),
            pltpu.VMEM((Q, K), jnp.int32),
        ],
        compiler_params=pltpu.CompilerParams(
            dimension_semantics=("arbitrary", "arbitrary"),
        ),
    )(query_seq, synth_set, na, nb)


def _gather_mean_body(synth_hbm, idx_hbm, out_hbm,
                      idx_v, rows0, rows1, out_v, sem0, sem1):
    wid = lax.axis_index("s") * NC + lax.axis_index("c")
    rows = (rows0, rows1)
    sems = (sem0, sem1)
    base = wid * RPW
    pltpu.sync_copy(idx_hbm.at[pl.ds(base * K, RPW * K)], idx_v)
    cps = []
    for c in range(NCH):
        cp = pltpu.async_copy(
            synth_hbm.at[idx_v.at[pl.ds(c * CR * K, CR * K)]],
            rows[c % 2], sems[c % 2])
        cps.append(cp)
        if c == 0:
            continue
        cps[c - 1].wait()
        _mean_chunk(rows[(c - 1) % 2], out_v)
        pltpu.sync_copy(out_v, out_hbm.at[pl.ds(base + (c - 1) * CR, CR)])
    cps[NCH - 1].wait()
    _mean_chunk(rows[(NCH - 1) % 2], out_v)
    pltpu.sync_copy(out_v, out_hbm.at[pl.ds(base + (NCH - 1) * CR, CR)])


def _mean_chunk(rows_v, out_v):
    def row_loop(i, carry):
        def col_loop(dd, c2):
            sl = pl.ds(dd * 16, 16)
            acc = (rows_v[K * i, sl] + rows_v[K * i + 1, sl]
                   + rows_v[K * i + 2, sl] + rows_v[K * i + 3, sl])
            out_v[i, sl] = acc * 0.25
            return c2
        return lax.fori_loop(0, D // 16, col_loop, carry)
    lax.fori_loop(0, CR, row_loop, 0)


def _gather_mean(synth_set, flat_idx):
    mesh = plsc.VectorSubcoreMesh(core_axis_name="c", subcore_axis_name="s")
    f = functools.partial(
        pl.kernel,
        out_type=jax.ShapeDtypeStruct((Q, D), jnp.float32),
        mesh=mesh,
        scratch_types=[
            pltpu.VMEM((RPW * K,), jnp.int32),
            pltpu.VMEM((CR * K, D), jnp.float32),
            pltpu.VMEM((CR * K, D), jnp.float32),
            pltpu.VMEM((CR, D), jnp.float32),
            pltpu.SemaphoreType.DMA,
            pltpu.SemaphoreType.DMA,
        ],
    )(_gather_mean_body)
    return f(synth_set, flat_idx)


def kernel(query_seq, synth_set, topk):
    del topk
    na = jnp.linalg.norm(query_seq, ord=2, axis=-1)[:, None]
    nb = jnp.linalg.norm(synth_set, ord=2, axis=-1)[None, :]
    idx = _topk_indices(query_seq, synth_set, na, nb)
    return _gather_mean(synth_set, idx.reshape(Q * K))


# speedup vs baseline: 3.0188x; 1.1018x over previous
import functools

import jax
import jax.numpy as jnp
from jax import lax
from jax.experimental import pallas as pl
from jax.experimental.pallas import tpu as pltpu
from jax.experimental.pallas import tpu_sc as plsc

Q, S, D, K = 2048, 16384, 1024, 4
QB, SB = 512, 2048

NC, NS = 2, 16
NW = NC * NS
RPW = Q // NW
CR = 8
NCH = RPW // CR

def _topk_body(q_ref, s_ref, na_ref, nb_ref, idx_ref, dists, idxs):
    si = pl.program_id(0)
    nsb = pl.num_programs(0)
    qi = pl.program_id(1)
    q = q_ref[...]
    s = s_ref[...]
    ab = lax.dot_general(q, s, (((1,), (1,)), ((), ())),
                         preferred_element_type=jnp.float32)
    na = na_ref[...]
    nb = nb_ref[...]
    na2 = na ** 2
    nb2 = nb ** 2
    sq = jnp.maximum(na2 + nb2 - 2.0 * ab, 0.0)
    dotprod = (-sq + na2 + nb2) / 2.0
    d = 1.0 - dotprod / (na * nb)

    qslice = pl.ds(qi * QB, QB)

    @pl.when(si == 0)
    def _():
        dists[qslice, :] = jnp.full((QB, K), jnp.inf, jnp.float32)
        idxs[qslice, :] = jnp.zeros((QB, K), jnp.int32)

    pos = lax.broadcasted_iota(jnp.int32, (QB, SB), 1) + si * SB
    comb_d = jnp.concatenate([dists[qslice, :], d], axis=1)
    comb_i = jnp.concatenate([idxs[qslice, :], pos], axis=1)
    new_d, new_i = [], []
    for _ in range(K):
        m = jnp.min(comb_d, axis=1)
        gi = jnp.min(jnp.where(comb_d == m[:, None], comb_i, 2**31 - 1), axis=1)
        new_d.append(m)
        new_i.append(gi)
        comb_d = jnp.where(comb_i == gi[:, None], jnp.inf, comb_d)
    dists[qslice, :] = jnp.concatenate([v[:, None] for v in new_d], axis=1)
    final_i = jnp.concatenate([i[:, None] for i in new_i], axis=1)
    idxs[qslice, :] = final_i
    idx_ref[...] = final_i


def _topk_indices(query_seq, synth_set, na, nb):
    return pl.pallas_call(
        _topk_body,
        grid=(S // SB, Q // QB),
        in_specs=[
            pl.BlockSpec((QB, D), lambda si, qi: (qi, 0)),
            pl.BlockSpec((SB, D), lambda si, qi: (si, 0)),
            pl.BlockSpec((QB, 1), lambda si, qi: (qi, 0)),
            pl.BlockSpec((1, SB), lambda si, qi: (0, si)),
        ],
        out_specs=pl.BlockSpec((QB, K), lambda si, qi: (qi, 0)),
        out_shape=jax.ShapeDtypeStruct((Q, K), jnp.int32),
        scratch_shapes=[
            pltpu.VMEM((Q, K), jnp.float32),
            pltpu.VMEM((Q, K), jnp.int32),
        ],
        compiler_params=pltpu.CompilerParams(
            dimension_semantics=("arbitrary", "arbitrary"),
        ),
    )(query_seq, synth_set, na, nb)


def _gather_mean_body(synth_hbm, idx_hbm, out_hbm,
                      idx_v, rows0, rows1, out_v, sem0, sem1):
    wid = lax.axis_index("s") * NC + lax.axis_index("c")
    rows = (rows0, rows1)
    sems = (sem0, sem1)
    base = wid * RPW
    pltpu.sync_copy(idx_hbm.at[pl.ds(base * K, RPW * K)], idx_v)
    cps = []
    for c in range(NCH):
        cp = pltpu.async_copy(
            synth_hbm.at[idx_v.at[pl.ds(c * CR * K, CR * K)]],
            rows[c % 2], sems[c % 2])
        cps.append(cp)
        if c == 0:
            continue
        cps[c - 1].wait()
        _mean_chunk(rows[(c - 1) % 2], out_v)
        pltpu.sync_copy(out_v, out_hbm.at[pl.ds(base + (c - 1) * CR, CR)])
    cps[NCH - 1].wait()
    _mean_chunk(rows[(NCH - 1) % 2], out_v)
    pltpu.sync_copy(out_v, out_hbm.at[pl.ds(base + (NCH - 1) * CR, CR)])


def _mean_chunk(rows_v, out_v):
    def row_loop(i, carry):
        def col_loop(dd, c2):
            sl = pl.ds(dd * 16, 16)
            acc = (rows_v[K * i, sl] + rows_v[K * i + 1, sl]
                   + rows_v[K * i + 2, sl] + rows_v[K * i + 3, sl])
            out_v[i, sl] = acc * 0.25
            return c2
        return lax.fori_loop(0, D // 16, col_loop, carry)
    lax.fori_loop(0, CR, row_loop, 0)


def _gather_mean(synth_set, flat_idx):
    mesh = plsc.VectorSubcoreMesh(core_axis_name="c", subcore_axis_name="s")
    f = functools.partial(
        pl.kernel,
        out_type=jax.ShapeDtypeStruct((Q, D), jnp.float32),
        mesh=mesh,
        scratch_types=[
            pltpu.VMEM((RPW * K,), jnp.int32),
            pltpu.VMEM((CR * K, D), jnp.float32),
            pltpu.VMEM((CR * K, D), jnp.float32),
            pltpu.VMEM((CR, D), jnp.float32),
            pltpu.SemaphoreType.DMA,
            pltpu.SemaphoreType.DMA,
        ],
    )(_gather_mean_body)
    return f(synth_set, flat_idx)


def kernel(query_seq, synth_set, topk):
    del topk
    na = jnp.linalg.norm(query_seq, ord=2, axis=-1)[:, None]
    nb = jnp.linalg.norm(synth_set, ord=2, axis=-1)[None, :]
    idx = _topk_indices(query_seq, synth_set, na, nb)
    return _gather_mean(synth_set, idx.reshape(Q * K))


# speedup vs baseline: 3.1615x; 1.0473x over previous
import functools

import jax
import jax.numpy as jnp
from jax import lax
from jax.experimental import pallas as pl
from jax.experimental.pallas import tpu as pltpu
from jax.experimental.pallas import tpu_sc as plsc

Q, S, D, K = 2048, 16384, 1024, 4
QB, SB = 1024, 2048

NC, NS = 2, 16
NW = NC * NS
RPW = Q // NW
CR = 8
NCH = RPW // CR

def _topk_body(q_ref, s_ref, na_ref, nb_ref, idx_ref, dists, idxs):
    si = pl.program_id(0)
    nsb = pl.num_programs(0)
    qi = pl.program_id(1)
    q = q_ref[...]
    s = s_ref[...]
    ab = lax.dot_general(q, s, (((1,), (1,)), ((), ())),
                         preferred_element_type=jnp.float32)
    na = na_ref[...]
    nb = nb_ref[...]
    na2 = na ** 2
    nb2 = nb ** 2
    sq = jnp.maximum(na2 + nb2 - 2.0 * ab, 0.0)
    dotprod = (-sq + na2 + nb2) / 2.0
    d = 1.0 - dotprod / (na * nb)

    qslice = pl.ds(qi * QB, QB)

    @pl.when(si == 0)
    def _():
        dists[qslice, :] = jnp.full((QB, K), jnp.inf, jnp.float32)
        idxs[qslice, :] = jnp.zeros((QB, K), jnp.int32)

    pos = lax.broadcasted_iota(jnp.int32, (QB, SB), 1) + si * SB
    comb_d = jnp.concatenate([dists[qslice, :], d], axis=1)
    comb_i = jnp.concatenate([idxs[qslice, :], pos], axis=1)
    new_d, new_i = [], []
    for _ in range(K):
        m = jnp.min(comb_d, axis=1)
        gi = jnp.min(jnp.where(comb_d == m[:, None], comb_i, 2**31 - 1), axis=1)
        new_d.append(m)
        new_i.append(gi)
        comb_d = jnp.where(comb_i == gi[:, None], jnp.inf, comb_d)
    dists[qslice, :] = jnp.concatenate([v[:, None] for v in new_d], axis=1)
    final_i = jnp.concatenate([i[:, None] for i in new_i], axis=1)
    idxs[qslice, :] = final_i
    idx_ref[...] = final_i


def _topk_indices(query_seq, synth_set, na, nb):
    return pl.pallas_call(
        _topk_body,
        grid=(S // SB, Q // QB),
        in_specs=[
            pl.BlockSpec((QB, D), lambda si, qi: (qi, 0)),
            pl.BlockSpec((SB, D), lambda si, qi: (si, 0)),
            pl.BlockSpec((QB, 1), lambda si, qi: (qi, 0)),
            pl.BlockSpec((1, SB), lambda si, qi: (0, si)),
        ],
        out_specs=pl.BlockSpec((QB, K), lambda si, qi: (qi, 0)),
        out_shape=jax.ShapeDtypeStruct((Q, K), jnp.int32),
        scratch_shapes=[
            pltpu.VMEM((Q, K), jnp.float32),
            pltpu.VMEM((Q, K), jnp.int32),
        ],
        compiler_params=pltpu.CompilerParams(
            dimension_semantics=("arbitrary", "arbitrary"),
        ),
    )(query_seq, synth_set, na, nb)


def _gather_mean_body(synth_hbm, idx_hbm, out_hbm,
                      idx_v, rows0, rows1, out_v, sem0, sem1):
    wid = lax.axis_index("s") * NC + lax.axis_index("c")
    rows = (rows0, rows1)
    sems = (sem0, sem1)
    base = wid * RPW
    pltpu.sync_copy(idx_hbm.at[pl.ds(base * K, RPW * K)], idx_v)
    cps = []
    for c in range(NCH):
        cp = pltpu.async_copy(
            synth_hbm.at[idx_v.at[pl.ds(c * CR * K, CR * K)]],
            rows[c % 2], sems[c % 2])
        cps.append(cp)
        if c == 0:
            continue
        cps[c - 1].wait()
        _mean_chunk(rows[(c - 1) % 2], out_v)
        pltpu.sync_copy(out_v, out_hbm.at[pl.ds(base + (c - 1) * CR, CR)])
    cps[NCH - 1].wait()
    _mean_chunk(rows[(NCH - 1) % 2], out_v)
    pltpu.sync_copy(out_v, out_hbm.at[pl.ds(base + (NCH - 1) * CR, CR)])


def _mean_chunk(rows_v, out_v):
    def row_loop(i, carry):
        def col_loop(dd, c2):
            sl = pl.ds(dd * 16, 16)
            acc = (rows_v[K * i, sl] + rows_v[K * i + 1, sl]
                   + rows_v[K * i + 2, sl] + rows_v[K * i + 3, sl])
            out_v[i, sl] = acc * 0.25
            return c2
        return lax.fori_loop(0, D // 16, col_loop, carry)
    lax.fori_loop(0, CR, row_loop, 0)


def _gather_mean(synth_set, flat_idx):
    mesh = plsc.VectorSubcoreMesh(core_axis_name="c", subcore_axis_name="s")
    f = functools.partial(
        pl.kernel,
        out_type=jax.ShapeDtypeStruct((Q, D), jnp.float32),
        mesh=mesh,
        scratch_types=[
            pltpu.VMEM((RPW * K,), jnp.int32),
            pltpu.VMEM((CR * K, D), jnp.float32),
            pltpu.VMEM((CR * K, D), jnp.float32),
            pltpu.VMEM((CR, D), jnp.float32),
            pltpu.SemaphoreType.DMA,
            pltpu.SemaphoreType.DMA,
        ],
    )(_gather_mean_body)
    return f(synth_set, flat_idx)


def kernel(query_seq, synth_set, topk):
    del topk
    na = jnp.linalg.norm(query_seq, ord=2, axis=-1)[:, None]
    nb = jnp.linalg.norm(synth_set, ord=2, axis=-1)[None, :]
    idx = _topk_indices(query_seq, synth_set, na, nb)
    return _gather_mean(synth_set, idx.reshape(Q * K))
